# baseline (speedup 1.0000x reference)
import functools

import jax
import jax.numpy as jnp
from jax import lax
from jax.experimental import pallas as pl
from jax.experimental.pallas import tpu as pltpu
from jax.experimental.pallas import tpu_sc as plsc

MAXLEN = 200
EMBED = 32
BATCH = 16384

NC = 2
NS = 16
NW = NC * NS

SEQ_PER_W = BATCH // NW
C = 4
NCHUNK = SEQ_PER_W // C
NBUF = 4
NBLK = NCHUNK // NBUF

_mesh = plsc.VectorSubcoreMesh(core_axis_name="c", subcore_axis_name="s")


@functools.partial(
    pl.kernel,
    mesh=_mesh,
    compiler_params=pltpu.CompilerParams(use_tc_tiling_on_sc=False),
    out_type=jax.ShapeDtypeStruct((BATCH, MAXLEN, EMBED), jnp.float32),
    scratch_types=[
        pltpu.VMEM((MAXLEN,), jnp.int32),
        pltpu.VMEM((NBUF, C, MAXLEN), jnp.int32),
        pltpu.VMEM((NBUF, C, MAXLEN, EMBED), jnp.float32),
        pltpu.SemaphoreType.DMA((NBUF,)),
        pltpu.SemaphoreType.DMA((NBUF,)),
        pltpu.SemaphoreType.DMA((NBUF,)),
    ],
)
def _embed_kernel(x_hbm, tok_hbm, pos_hbm, pid_hbm, out_hbm,
                  pidx_v, idx_v, buf, gsem, asem, osem):
    wid = lax.axis_index("s") * NC + lax.axis_index("c")
    pltpu.sync_copy(pid_hbm, pidx_v)

    def stage_and_fire(g, k):
        seqb = wid * SEQ_PER_W + g * C
        pltpu.sync_copy(x_hbm.at[pl.ds(seqb, C)], idx_v.at[k])
        for s in range(C):
            pltpu.async_copy(
                tok_hbm.at[idx_v.at[k].at[s]], buf.at[k].at[s], gsem.at[k])

    def fire_pos_add(k):
        for s in range(C):
            pltpu.async_copy(
                pos_hbm.at[pidx_v], buf.at[k].at[s], asem.at[k], add=True)

    def wait_sem(sem_k):
        pltpu.make_async_copy(
            out_hbm.at[pl.ds(0, C)], buf.at[0], sem_k).wait()

    def wait_out(k):
        pltpu.make_async_copy(
            buf.at[0], out_hbm.at[pl.ds(0, C)], osem.at[k]).wait()

    stage_and_fire(0, 0)

    def block_body(b, carry):
        for k in range(NBUF):
            g = b * NBUF + k
            k1 = (k + 1) % NBUF
            gnext = g + 1

            @pl.when(jnp.logical_and(gnext < NCHUNK, gnext >= NBUF))
            def _():
                wait_out(k1)

            @pl.when(gnext < NCHUNK)
            def _():
                stage_and_fire(gnext, k1)

            wait_sem(gsem.at[k])
            fire_pos_add(k)
            wait_sem(asem.at[k])
            pltpu.async_copy(
                buf.at[k],
                out_hbm.at[pl.ds(wid * SEQ_PER_W + g * C, C)],
                osem.at[k],
            )
        return carry

    lax.fori_loop(0, NBLK, block_body, 0)
    for k in range(NBUF):
        wait_out(k)


def kernel(x, token_table, pos_table):
    pos_ids = jnp.arange(MAXLEN, dtype=jnp.int32)
    return _embed_kernel(
        x.astype(jnp.int32), token_table, pos_table, pos_ids)


# speedup vs baseline: 1.4420x; 1.4420x over previous
import functools

import jax
import jax.numpy as jnp
from jax import lax
from jax.experimental import pallas as pl
from jax.experimental.pallas import tpu as pltpu
from jax.experimental.pallas import tpu_sc as plsc

MAXLEN = 200
EMBED = 32
BATCH = 16384

NC = 2
NS = 16
NW = NC * NS

SEQ_PER_W = BATCH // NW
C = 4
ROWS = C * MAXLEN
NCHUNK = SEQ_PER_W // C
GSUB = 100
NG = ROWS // GSUB
XROW_PER_W = (SEQ_PER_W * MAXLEN) // GSUB
ROW_PER_W = SEQ_PER_W * MAXLEN
NBUF = 4
NBLK = NCHUNK // NBUF

_mesh = plsc.VectorSubcoreMesh(core_axis_name="c", subcore_axis_name="s")


@functools.partial(
    pl.kernel,
    mesh=_mesh,
    compiler_params=pltpu.CompilerParams(use_tc_tiling_on_sc=False),
    out_type=jax.ShapeDtypeStruct((BATCH * MAXLEN, EMBED), jnp.float32),
    scratch_types=[
        pltpu.VMEM((NBUF, NG, GSUB), jnp.int32),
        pltpu.VMEM((NBUF, ROWS, EMBED), jnp.float32),
        pltpu.VMEM((MAXLEN, EMBED), jnp.float32),
        pltpu.SemaphoreType.DMA((NBUF,)),
        pltpu.SemaphoreType.DMA((NBUF,)),
    ],
)
def _embed_kernel(x_hbm, tok_hbm, pos_hbm, out_hbm, idx_v, buf, pos_v,
                  gsem, osem):
    wid = lax.axis_index("s") * NC + lax.axis_index("c")
    pltpu.sync_copy(pos_hbm, pos_v)

    def stage_and_fire(g, k):
        pltpu.sync_copy(
            x_hbm.at[pl.ds(wid * XROW_PER_W + g * NG, NG)], idx_v.at[k])
        for j in range(NG):
            pltpu.async_copy(
                tok_hbm.at[idx_v.at[k].at[j]],
                buf.at[k].at[pl.ds(j * GSUB, GSUB)],
                gsem.at[k],
            )

    def wait_gather(k):
        pltpu.make_async_copy(
            out_hbm.at[pl.ds(0, ROWS)], buf.at[k], gsem.at[k]).wait()

    def wait_out(k):
        pltpu.make_async_copy(
            buf.at[k], out_hbm.at[pl.ds(0, ROWS)], osem.at[k]).wait()

    stage_and_fire(0, 0)

    def block_body(b, carry):
        for k in range(NBUF):
            g = b * NBUF + k
            k1 = (k + 1) % NBUF
            gnext = g + 1

            @pl.when(jnp.logical_and(gnext < NCHUNK, gnext >= NBUF))
            def _():
                wait_out(k1)

            @pl.when(gnext < NCHUNK)
            def _():
                stage_and_fire(gnext, k1)

            wait_gather(k)

            def add_body(t, c2):
                p0 = pos_v[t, pl.ds(0, 16)]
                p1 = pos_v[t, pl.ds(16, 16)]
                for s in range(C):
                    r = s * MAXLEN + t
                    buf[k, r, pl.ds(0, 16)] += p0
                    buf[k, r, pl.ds(16, 16)] += p1
                return c2

            lax.fori_loop(0, MAXLEN, add_body, 0)
            pltpu.async_copy(
                buf.at[k],
                out_hbm.at[pl.ds(wid * ROW_PER_W + g * ROWS, ROWS)],
                osem.at[k],
            )
        return carry

    lax.fori_loop(0, NBLK, block_body, 0)
    for k in range(NBUF):
        wait_out(k)


def kernel(x, token_table, pos_table):
    x_flat = x.reshape(-1).astype(jnp.int32).reshape(-1, GSUB)
    out = _embed_kernel(x_flat, token_table, pos_table)
    return out.reshape(BATCH, MAXLEN, EMBED)


# speedup vs baseline: 2.2976x; 1.5934x over previous
import functools

import jax
import jax.numpy as jnp
from jax import lax
from jax.experimental import pallas as pl
from jax.experimental.pallas import tpu as pltpu
from jax.experimental.pallas import tpu_sc as plsc

MAXLEN = 200
EMBED = 32
BATCH = 16384

NC = 2
NS = 16
NW = NC * NS

SEQ_PER_W = BATCH // NW
C = 4
ROWS = C * MAXLEN
NCHUNK = SEQ_PER_W // C
GSUB = 100
NG = ROWS // GSUB
XROW_PER_W = (SEQ_PER_W * MAXLEN) // GSUB
ROW_PER_W = SEQ_PER_W * MAXLEN
NBUF = 4
NBLK = NCHUNK // NBUF

_mesh = plsc.VectorSubcoreMesh(core_axis_name="c", subcore_axis_name="s")


@functools.partial(
    pl.kernel,
    mesh=_mesh,
    compiler_params=pltpu.CompilerParams(use_tc_tiling_on_sc=False),
    out_type=jax.ShapeDtypeStruct((BATCH * MAXLEN, EMBED), jnp.float32),
    scratch_types=[
        pltpu.VMEM((NBUF, NG, GSUB), jnp.int32),
        pltpu.VMEM((NBUF, ROWS, EMBED), jnp.float32),
        pltpu.SemaphoreType.DMA((NBUF,)),
        pltpu.SemaphoreType.DMA((NBUF,)),
    ],
)
def _gather_kernel(x_hbm, tok_hbm, out_hbm, idx_v, buf, gsem, osem):
    wid = lax.axis_index("s") * NC + lax.axis_index("c")

    def stage_and_fire(g, k):
        pltpu.sync_copy(
            x_hbm.at[pl.ds(wid * XROW_PER_W + g * NG, NG)], idx_v.at[k])
        for j in range(NG):
            pltpu.async_copy(
                tok_hbm.at[idx_v.at[k].at[j]],
                buf.at[k].at[pl.ds(j * GSUB, GSUB)],
                gsem.at[k],
            )

    def wait_gather(k):
        pltpu.make_async_copy(
            out_hbm.at[pl.ds(0, ROWS)], buf.at[k], gsem.at[k]).wait()

    def wait_out(k):
        pltpu.make_async_copy(
            buf.at[k], out_hbm.at[pl.ds(0, ROWS)], osem.at[k]).wait()

    stage_and_fire(0, 0)

    def block_body(b, carry):
        for k in range(NBUF):
            g = b * NBUF + k
            k1 = (k + 1) % NBUF
            gnext = g + 1

            @pl.when(jnp.logical_and(gnext < NCHUNK, gnext >= NBUF))
            def _():
                wait_out(k1)

            @pl.when(gnext < NCHUNK)
            def _():
                stage_and_fire(gnext, k1)

            wait_gather(k)
            pltpu.async_copy(
                buf.at[k],
                out_hbm.at[pl.ds(wid * ROW_PER_W + g * ROWS, ROWS)],
                osem.at[k],
            )
        return carry

    lax.fori_loop(0, NBLK, block_body, 0)
    for k in range(NBUF):
        wait_out(k)


BBLK = 128
T4 = MAXLEN // 4


def _tc_body(i_ref, pos_ref, o_ref):
    blk = i_ref[...]
    blk = blk.reshape(BBLK, T4, 128).transpose(1, 2, 0)
    blk = blk.reshape(MAXLEN, EMBED, BBLK)
    o_ref[...] = blk + pos_ref[...][:, :, None]


_tc_relayout = pl.pallas_call(
    _tc_body,
    grid=(BATCH // BBLK,),
    in_specs=[
        pl.BlockSpec((BBLK * T4, 128), lambda i: (i, 0)),
        pl.BlockSpec((MAXLEN, EMBED), lambda i: (0, 0)),
    ],
    out_specs=pl.BlockSpec((MAXLEN, EMBED, BBLK), lambda i: (0, 0, i)),
    out_shape=jax.ShapeDtypeStruct((MAXLEN, EMBED, BATCH), jnp.float32),
)


def kernel(x, token_table, pos_table):
    x_flat = x.reshape(-1).astype(jnp.int32).reshape(-1, GSUB)
    tok = _gather_kernel(x_flat, token_table)
    tok128 = tok.reshape(BATCH * T4, 128)
    out_tdb = _tc_relayout(tok128, pos_table)
    return out_tdb.transpose(2, 0, 1)
